# baseline (speedup 1.0000x reference)
import jax
import jax.numpy as jnp
from jax.experimental import pallas as pl

_V = 1000
_D = 16
_B = 1024
_VB = 250


def _fill_kernel(x_ref, w0_ref, w1_ref, o_ref):
    i = pl.program_id(0)
    xv = x_ref[...]
    viota = jax.lax.broadcasted_iota(jnp.int32, (_VB, _D, _B), 0) + i * _VB
    mask = viota == xv
    base = w0_ref[...]
    alt = w1_ref[...]
    o_ref[...] = jnp.where(mask, alt, base)


def kernel(x, W):
    x3 = x.astype(jnp.int32).reshape(1, 1, _B)
    w0 = W[0].reshape(1, _D, 1)
    w1 = W[1].reshape(1, _D, 1)
    out_t = pl.pallas_call(
        _fill_kernel,
        grid=(_V // _VB,),
        in_specs=[
            pl.BlockSpec((1, 1, _B), lambda i: (0, 0, 0)),
            pl.BlockSpec((1, _D, 1), lambda i: (0, 0, 0)),
            pl.BlockSpec((1, _D, 1), lambda i: (0, 0, 0)),
        ],
        out_specs=pl.BlockSpec((_VB, _D, _B), lambda i: (i, 0, 0)),
        out_shape=jax.ShapeDtypeStruct((_V, _D, _B), jnp.float32),
    )(x3, w0, w1)
    return jnp.transpose(out_t, (2, 0, 1))


# speedup vs baseline: 1.0721x; 1.0721x over previous
import jax
import jax.numpy as jnp
from jax.experimental import pallas as pl

_V = 1000
_D = 16
_B = 1024
_VB = 100


def _fill_kernel(x_ref, w0_ref, w1_ref, o_ref):
    i = pl.program_id(0)
    xv = x_ref[...]
    viota = jax.lax.broadcasted_iota(jnp.int32, (_VB, _D, _B), 0) + i * _VB
    mask = viota == xv
    base = w0_ref[...]
    alt = w1_ref[...]
    o_ref[...] = jnp.where(mask, alt, base)


def kernel(x, W):
    x3 = x.astype(jnp.int32).reshape(1, 1, _B)
    w0 = W[0].reshape(1, _D, 1)
    w1 = W[1].reshape(1, _D, 1)
    out_t = pl.pallas_call(
        _fill_kernel,
        grid=(_V // _VB,),
        in_specs=[
            pl.BlockSpec((1, 1, _B), lambda i: (0, 0, 0)),
            pl.BlockSpec((1, _D, 1), lambda i: (0, 0, 0)),
            pl.BlockSpec((1, _D, 1), lambda i: (0, 0, 0)),
        ],
        out_specs=pl.BlockSpec((_VB, _D, _B), lambda i: (i, 0, 0)),
        out_shape=jax.ShapeDtypeStruct((_V, _D, _B), jnp.float32),
    )(x3, w0, w1)
    return jnp.transpose(out_t, (2, 0, 1))
